# baseline (speedup 1.0000x reference)
import functools
import jax
import jax.numpy as jnp
from jax import lax
from jax.experimental import pallas as pl
from jax.experimental.pallas import tpu as pltpu
from jax.experimental.pallas import tpu_sc as plsc

B = 4096
PEP_LEN = 21
MHC_LEN = 34
CORE_LEN = 15
EMB = 128
PAD = 3
VOCAB = 30

NC = 2
NS = 16
NW = NC * NS

ROWS_W = B // NW
PEP_W = ROWS_W * PEP_LEN
MHC_W = ROWS_W * MHC_LEN
MSK_W = ROWS_W * CORE_LEN
CHUNK = 128
PEP_CHUNKS = PEP_W // CHUNK
MHC_CHUNKS = MHC_W // CHUNK
NBUF = 5
PREF = 3
MSK_VECS = MSK_W // 16

_mesh = plsc.VectorSubcoreMesh(core_axis_name="c", subcore_axis_name="s")


@functools.partial(
    pl.kernel,
    mesh=_mesh,
    out_type=[
        jax.ShapeDtypeStruct((B * PEP_LEN, EMB), jnp.float32),
        jax.ShapeDtypeStruct((B * MHC_LEN, EMB), jnp.float32),
        jax.ShapeDtypeStruct((B * CORE_LEN,), jnp.int32),
    ],
    scratch_types=[
        pltpu.VMEM((PEP_W,), jnp.int32),
        pltpu.VMEM((MHC_W,), jnp.int32),
        pltpu.VMEM((ROWS_W, PEP_LEN), jnp.int32),
        pltpu.VMEM((ROWS_W, MHC_LEN), jnp.int32),
        pltpu.VMEM((MSK_W + 16,), jnp.int32),
        pltpu.VMEM_SHARED((VOCAB, EMB), jnp.float32),
        pltpu.VMEM_SHARED((VOCAB, EMB), jnp.float32),
    ]
    + [pltpu.VMEM((CHUNK, EMB), jnp.float32) for _ in range(NBUF)]
    + [pltpu.SemaphoreType.DMA for _ in range(2 * NBUF + 1)],
)
def _emb_lookup(pep_x2, mhc_x2, pep_tab, mhc_tab,
                pep_out, mhc_out, msk_out,
                pep_idx_v, mhc_idx_v, pep_idx2_v, mhc_idx2_v,
                msk_v, pep_tab_v, mhc_tab_v,
                *bufs_and_sems):
    bufs = list(bufs_and_sems[:NBUF])
    gsems = list(bufs_and_sems[NBUF:2 * NBUF])
    osems = list(bufs_and_sems[2 * NBUF:3 * NBUF])
    msem = bufs_and_sems[3 * NBUF]

    wid = lax.axis_index("s") * NC + lax.axis_index("c")

    pltpu.sync_copy(pep_x2.at[pl.ds(wid * ROWS_W, ROWS_W)], pep_idx2_v)
    pltpu.sync_copy(mhc_x2.at[pl.ds(wid * ROWS_W, ROWS_W)], mhc_idx2_v)

    @pl.when(lax.axis_index("s") == 0)
    def _stage_tables():
        pltpu.sync_copy(pep_tab, pep_tab_v)
        pltpu.sync_copy(mhc_tab, mhc_tab_v)

    plsc.subcore_barrier()

    def flat_body(b, carry):
        p0 = pep_idx2_v[b, pl.ds(0, 16)]
        p1 = pep_idx2_v[b, pl.ds(PEP_LEN - 16, 16)]
        pep_idx_v[pl.ds(b * PEP_LEN, 16)] = p0
        pep_idx_v[pl.ds(b * PEP_LEN + PEP_LEN - 16, 16)] = p1
        m0 = mhc_idx2_v[b, pl.ds(0, 16)]
        m1 = mhc_idx2_v[b, pl.ds(16, 16)]
        m2 = mhc_idx2_v[b, pl.ds(MHC_LEN - 16, 16)]
        mhc_idx_v[pl.ds(b * MHC_LEN, 16)] = m0
        mhc_idx_v[pl.ds(b * MHC_LEN + 16, 16)] = m1
        mhc_idx_v[pl.ds(b * MHC_LEN + MHC_LEN - 16, 16)] = m2
        mv = pep_idx2_v[b, pl.ds(PAD, 16)]
        m = jnp.where(mv != jnp.zeros((16,), jnp.int32),
                      jnp.ones((16,), jnp.int32),
                      jnp.zeros((16,), jnp.int32))
        msk_v[pl.ds(b * CORE_LEN, 16)] = m
        return carry

    lax.fori_loop(0, ROWS_W, flat_body, 0)
    mcopy = pltpu.async_copy(msk_v.at[pl.ds(0, MSK_W)],
                             msk_out.at[pl.ds(wid * MSK_W, MSK_W)], msem)

    def gather(idx_v, tab, c):
        s = c % NBUF
        return pltpu.async_copy(
            tab.at[idx_v.at[pl.ds(c * CHUNK, CHUNK)]], bufs[s], gsems[s])

    def prime(idx_v, tab, nchunks):
        g = [None] * NBUF
        for c in range(min(PREF, nchunks)):
            g[c % NBUF] = gather(idx_v, tab, c)
        return g

    def drain(g, idx_v, tab, out, base, nchunks):
        o = [None] * NBUF
        pending = [False] * NBUF
        for j in range(nchunks):
            s = j % NBUF
            if g[s] is not None:
                g[s].wait()
            o[s] = pltpu.async_copy(
                bufs[s], out.at[pl.ds(base + j * CHUNK, CHUNK)], osems[s])
            pending[s] = True
            c = j + PREF
            if c < nchunks:
                sc = c % NBUF
                if pending[sc]:
                    o[sc].wait()
                    pending[sc] = False
                g[sc] = gather(idx_v, tab, c)
        for s in range(NBUF):
            if pending[s]:
                o[s].wait()

    g = prime(pep_idx_v, pep_tab_v, PEP_CHUNKS)
    drain(g, pep_idx_v, pep_tab_v, pep_out, wid * PEP_W, PEP_CHUNKS)

    g = prime(mhc_idx_v, mhc_tab_v, MHC_CHUNKS)
    drain(g, mhc_idx_v, mhc_tab_v, mhc_out, wid * MHC_W, MHC_CHUNKS)

    mcopy.wait()


def kernel(peptide_x, peptide_esm_x, mhc_x, peptide_emb, mhc_emb):
    del peptide_esm_x
    pep_out, mhc_out, msk = _emb_lookup(peptide_x.astype(jnp.int32),
                                        mhc_x.astype(jnp.int32),
                                        peptide_emb, mhc_emb)
    peptide_out = pep_out.reshape(B, PEP_LEN, EMB)
    mhc_out = mhc_out.reshape(B, MHC_LEN, EMB)
    masks = msk.reshape(B, CORE_LEN).astype(jnp.bool_)
    return (peptide_out, masks, mhc_out)


# speedup vs baseline: 1.6525x; 1.6525x over previous
import functools
import jax
import jax.numpy as jnp
from jax import lax
from jax.experimental import pallas as pl
from jax.experimental.pallas import tpu as pltpu
from jax.experimental.pallas import tpu_sc as plsc

B = 4096
PEP_LEN = 21
MHC_LEN = 34
CORE_LEN = 15
EMB = 128
PAD = 3
VOCAB = 30

NC = 2
NS = 16
NW = NC * NS

ROWS_W = B // NW
MSK_W = ROWS_W * CORE_LEN

GP = 4
GM = 2
NBUF = 4
PEP_ITERS = ROWS_W // (GP * NBUF)
MHC_ITERS = ROWS_W // (GM * NBUF)

_mesh = plsc.VectorSubcoreMesh(core_axis_name="c", subcore_axis_name="s")


@functools.partial(
    pl.kernel,
    mesh=_mesh,
    out_type=[
        jax.ShapeDtypeStruct((B, PEP_LEN, EMB), jnp.float32),
        jax.ShapeDtypeStruct((B, MHC_LEN, EMB), jnp.float32),
        jax.ShapeDtypeStruct((B * CORE_LEN,), jnp.int32),
    ],
    scratch_types=[
        pltpu.VMEM((ROWS_W, PEP_LEN), jnp.int32),
        pltpu.VMEM((ROWS_W, MHC_LEN), jnp.int32),
        pltpu.VMEM((MSK_W + 16,), jnp.int32),
        pltpu.VMEM_SHARED((VOCAB, EMB), jnp.float32),
        pltpu.VMEM_SHARED((VOCAB, EMB), jnp.float32),
    ]
    + [pltpu.VMEM((GP, PEP_LEN, EMB), jnp.float32) for _ in range(NBUF)]
    + [pltpu.VMEM((GM, MHC_LEN, EMB), jnp.float32) for _ in range(NBUF)]
    + [pltpu.SemaphoreType.DMA for _ in range(2 * NBUF + 1)],
)
def _emb_lookup(pep_x2, mhc_x2, pep_tab, mhc_tab,
                pep_out, mhc_out, msk_out,
                pep_idx2_v, mhc_idx2_v, msk_v, pep_tab_v, mhc_tab_v,
                *bufs_and_sems):
    pbufs = list(bufs_and_sems[:NBUF])
    mbufs = list(bufs_and_sems[NBUF:2 * NBUF])
    gsems = list(bufs_and_sems[2 * NBUF:3 * NBUF])
    osems = list(bufs_and_sems[3 * NBUF:4 * NBUF])
    msem = bufs_and_sems[4 * NBUF]

    wid = lax.axis_index("s") * NC + lax.axis_index("c")
    row0 = wid * ROWS_W

    pltpu.sync_copy(pep_x2.at[pl.ds(row0, ROWS_W)], pep_idx2_v)
    pltpu.sync_copy(mhc_x2.at[pl.ds(row0, ROWS_W)], mhc_idx2_v)

    @pl.when(lax.axis_index("s") == 0)
    def _stage_tables():
        pltpu.sync_copy(pep_tab, pep_tab_v)
        pltpu.sync_copy(mhc_tab, mhc_tab_v)

    plsc.subcore_barrier()

    def mask_body(b, carry):
        mv = pep_idx2_v[b, pl.ds(PAD, 16)]
        m = jnp.where(mv != jnp.zeros((16,), jnp.int32),
                      jnp.ones((16,), jnp.int32),
                      jnp.zeros((16,), jnp.int32))
        msk_v[pl.ds(b * CORE_LEN, 16)] = m
        return carry

    lax.fori_loop(0, ROWS_W, mask_body, 0)
    mcopy = pltpu.async_copy(msk_v.at[pl.ds(0, MSK_W)],
                             msk_out.at[pl.ds(wid * MSK_W, MSK_W)], msem)

    def phase(idx2_v, tab_v, out, bufs, group, niters):
        def body(gg, carry):
            g0 = gg * NBUF
            gcs = []
            for s in range(NBUF):
                b0 = (g0 + s) * group
                cps = []
                for r in range(group):
                    cps.append(pltpu.async_copy(
                        tab_v.at[idx2_v.at[b0 + r]], bufs[s].at[r],
                        gsems[s]))
                gcs.append(cps)
            ocs = []
            for s in range(NBUF):
                for cp in gcs[s]:
                    cp.wait()
                b0 = (g0 + s) * group
                ocs.append(pltpu.async_copy(
                    bufs[s], out.at[pl.ds(row0 + b0, group)], osems[s]))
            for oc in ocs:
                oc.wait()
            return carry

        lax.fori_loop(0, niters, body, 0)

    phase(pep_idx2_v, pep_tab_v, pep_out, pbufs, GP, PEP_ITERS)
    phase(mhc_idx2_v, mhc_tab_v, mhc_out, mbufs, GM, MHC_ITERS)

    mcopy.wait()


def kernel(peptide_x, peptide_esm_x, mhc_x, peptide_emb, mhc_emb):
    del peptide_esm_x
    pep_out, mhc_out, msk = _emb_lookup(peptide_x.astype(jnp.int32),
                                        mhc_x.astype(jnp.int32),
                                        peptide_emb, mhc_emb)
    masks = msk.reshape(B, CORE_LEN).astype(jnp.bool_)
    return (pep_out, masks, mhc_out)


# speedup vs baseline: 1.6533x; 1.0005x over previous
import functools
import jax
import jax.numpy as jnp
from jax import lax
from jax.experimental import pallas as pl
from jax.experimental.pallas import tpu as pltpu
from jax.experimental.pallas import tpu_sc as plsc

B = 4096
PEP_LEN = 21
MHC_LEN = 34
CORE_LEN = 15
EMB = 128
PAD = 3
VOCAB = 30

NC = 2
NS = 16
NW = NC * NS

ROWS_W = B // NW
MSK_W = ROWS_W * CORE_LEN

GP = 4
GM = 2
NBUF = 4
PEP_ITERS = ROWS_W // (GP * NBUF)
MHC_ITERS = ROWS_W // (GM * NBUF)

_mesh = plsc.VectorSubcoreMesh(core_axis_name="c", subcore_axis_name="s")


@functools.partial(
    pl.kernel,
    mesh=_mesh,
    out_type=[
        jax.ShapeDtypeStruct((B, PEP_LEN, EMB), jnp.float32),
        jax.ShapeDtypeStruct((B, MHC_LEN, EMB), jnp.float32),
        jax.ShapeDtypeStruct((B * CORE_LEN,), jnp.int32),
    ],
    scratch_types=[
        pltpu.VMEM((ROWS_W, PEP_LEN), jnp.int32),
        pltpu.VMEM((ROWS_W, MHC_LEN), jnp.int32),
        pltpu.VMEM((MSK_W + 16,), jnp.int32),
        pltpu.VMEM_SHARED((VOCAB, EMB), jnp.float32),
        pltpu.VMEM_SHARED((VOCAB, EMB), jnp.float32),
    ]
    + [pltpu.VMEM((GP, PEP_LEN, EMB), jnp.float32) for _ in range(NBUF)]
    + [pltpu.VMEM((GM, MHC_LEN, EMB), jnp.float32) for _ in range(NBUF)]
    + [pltpu.SemaphoreType.DMA for _ in range(2 * NBUF + 1)],
    compiler_params=pltpu.CompilerParams(use_tc_tiling_on_sc=True),
)
def _emb_lookup(pep_x2, mhc_x2, pep_tab, mhc_tab,
                pep_out, mhc_out, msk_out,
                pep_idx2_v, mhc_idx2_v, msk_v, pep_tab_v, mhc_tab_v,
                *bufs_and_sems):
    pbufs = list(bufs_and_sems[:NBUF])
    mbufs = list(bufs_and_sems[NBUF:2 * NBUF])
    gsems = list(bufs_and_sems[2 * NBUF:3 * NBUF])
    osems = list(bufs_and_sems[3 * NBUF:4 * NBUF])
    msem = bufs_and_sems[4 * NBUF]

    wid = lax.axis_index("s") * NC + lax.axis_index("c")
    row0 = wid * ROWS_W

    pltpu.sync_copy(pep_x2.at[pl.ds(row0, ROWS_W)], pep_idx2_v)
    pltpu.sync_copy(mhc_x2.at[pl.ds(row0, ROWS_W)], mhc_idx2_v)

    @pl.when(lax.axis_index("s") == 0)
    def _stage_tables():
        pltpu.sync_copy(pep_tab, pep_tab_v)
        pltpu.sync_copy(mhc_tab, mhc_tab_v)

    plsc.subcore_barrier()

    def mask_body(b, carry):
        mv = pep_idx2_v[b, pl.ds(PAD, 16)]
        m = jnp.where(mv != jnp.zeros((16,), jnp.int32),
                      jnp.ones((16,), jnp.int32),
                      jnp.zeros((16,), jnp.int32))
        msk_v[pl.ds(b * CORE_LEN, 16)] = m
        return carry

    lax.fori_loop(0, ROWS_W, mask_body, 0)
    mcopy = pltpu.async_copy(msk_v.at[pl.ds(0, MSK_W)],
                             msk_out.at[pl.ds(wid * MSK_W, MSK_W)], msem)

    def phase(idx2_v, tab_v, out, bufs, group, niters):
        def body(gg, carry):
            g0 = gg * NBUF
            gcs = []
            for s in range(NBUF):
                b0 = (g0 + s) * group
                cps = []
                for r in range(group):
                    cps.append(pltpu.async_copy(
                        tab_v.at[idx2_v.at[b0 + r]], bufs[s].at[r],
                        gsems[s]))
                gcs.append(cps)
            ocs = []
            for s in range(NBUF):
                for cp in gcs[s]:
                    cp.wait()
                b0 = (g0 + s) * group
                ocs.append(pltpu.async_copy(
                    bufs[s], out.at[pl.ds(row0 + b0, group)], osems[s]))
            for oc in ocs:
                oc.wait()
            return carry

        lax.fori_loop(0, niters, body, 0)

    phase(pep_idx2_v, pep_tab_v, pep_out, pbufs, GP, PEP_ITERS)
    phase(mhc_idx2_v, mhc_tab_v, mhc_out, mbufs, GM, MHC_ITERS)

    mcopy.wait()


def kernel(peptide_x, peptide_esm_x, mhc_x, peptide_emb, mhc_emb):
    del peptide_esm_x
    pep_out, mhc_out, msk = _emb_lookup(peptide_x.astype(jnp.int32),
                                        mhc_x.astype(jnp.int32),
                                        peptide_emb, mhc_emb)
    masks = msk.reshape(B, CORE_LEN).astype(jnp.bool_)
    return (pep_out, masks, mhc_out)


# speedup vs baseline: 1.6861x; 1.0198x over previous
import functools
import jax
import jax.numpy as jnp
from jax import lax
from jax.experimental import pallas as pl
from jax.experimental.pallas import tpu as pltpu
from jax.experimental.pallas import tpu_sc as plsc

B = 4096
PEP_LEN = 21
MHC_LEN = 34
CORE_LEN = 15
EMB = 128
PAD = 3
VOCAB = 30

NC = 2
NS = 16
NW = NC * NS

ROWS_W = B // NW
MSK_W = ROWS_W * CORE_LEN

GP = 4
NBUF = 4
PEP_ITERS = ROWS_W // (GP * NBUF)

_mesh = plsc.VectorSubcoreMesh(core_axis_name="c", subcore_axis_name="s")


@functools.partial(
    pl.kernel,
    mesh=_mesh,
    out_type=[
        jax.ShapeDtypeStruct((B, PEP_LEN, EMB), jnp.float32),
        jax.ShapeDtypeStruct((B * CORE_LEN,), jnp.int32),
    ],
    scratch_types=[
        pltpu.VMEM((ROWS_W, PEP_LEN), jnp.int32),
        pltpu.VMEM((MSK_W + 16,), jnp.int32),
        pltpu.VMEM_SHARED((VOCAB, EMB), jnp.float32),
    ]
    + [pltpu.VMEM((GP, PEP_LEN, EMB), jnp.float32) for _ in range(NBUF)]
    + [pltpu.SemaphoreType.DMA for _ in range(2 * NBUF + 1)],
)
def _pep_lookup(pep_x2, pep_tab, pep_out, msk_out,
                pep_idx2_v, msk_v, pep_tab_v, *bufs_and_sems):
    bufs = list(bufs_and_sems[:NBUF])
    gsems = list(bufs_and_sems[NBUF:2 * NBUF])
    osems = list(bufs_and_sems[2 * NBUF:3 * NBUF])
    msem = bufs_and_sems[3 * NBUF]

    wid = lax.axis_index("s") * NC + lax.axis_index("c")
    row0 = wid * ROWS_W

    pltpu.sync_copy(pep_x2.at[pl.ds(row0, ROWS_W)], pep_idx2_v)

    @pl.when(lax.axis_index("s") == 0)
    def _stage_table():
        pltpu.sync_copy(pep_tab, pep_tab_v)

    plsc.subcore_barrier()

    def mask_body(b, carry):
        mv = pep_idx2_v[b, pl.ds(PAD, 16)]
        m = jnp.where(mv != jnp.zeros((16,), jnp.int32),
                      jnp.ones((16,), jnp.int32),
                      jnp.zeros((16,), jnp.int32))
        msk_v[pl.ds(b * CORE_LEN, 16)] = m
        return carry

    lax.fori_loop(0, ROWS_W, mask_body, 0)
    mcopy = pltpu.async_copy(msk_v.at[pl.ds(0, MSK_W)],
                             msk_out.at[pl.ds(wid * MSK_W, MSK_W)], msem)

    def body(gg, carry):
        g0 = gg * NBUF
        gcs = []
        for s in range(NBUF):
            b0 = (g0 + s) * GP
            cps = []
            for r in range(GP):
                cps.append(pltpu.async_copy(
                    pep_tab_v.at[pep_idx2_v.at[b0 + r]], bufs[s].at[r],
                    gsems[s]))
            gcs.append(cps)
        ocs = []
        for s in range(NBUF):
            for cp in gcs[s]:
                cp.wait()
            b0 = (g0 + s) * GP
            ocs.append(pltpu.async_copy(
                bufs[s], pep_out.at[pl.ds(row0 + b0, GP)], osems[s]))
        for oc in ocs:
            oc.wait()
        return carry

    lax.fori_loop(0, PEP_ITERS, body, 0)
    mcopy.wait()


BB = 128


def _mhc_body(x_ref, tab_ref, o_ref):
    x = x_ref[...]
    iota = lax.broadcasted_iota(jnp.int32, (1, 1, VOCAB), 2)
    oh = (x[:, :, None] == iota).astype(jnp.float32)
    res = jnp.dot(oh.reshape(BB * MHC_LEN, VOCAB), tab_ref[...],
                  preferred_element_type=jnp.float32)
    o_ref[...] = res.reshape(BB, MHC_LEN, EMB)


_mhc_matmul = pl.pallas_call(
    _mhc_body,
    grid=(B // BB,),
    in_specs=[
        pl.BlockSpec((BB, MHC_LEN), lambda i: (i, 0)),
        pl.BlockSpec((VOCAB, EMB), lambda i: (0, 0)),
    ],
    out_specs=pl.BlockSpec((BB, MHC_LEN, EMB), lambda i: (i, 0, 0)),
    out_shape=jax.ShapeDtypeStruct((B, MHC_LEN, EMB), jnp.float32),
)


def kernel(peptide_x, peptide_esm_x, mhc_x, peptide_emb, mhc_emb):
    del peptide_esm_x
    pep_out, msk = _pep_lookup(peptide_x.astype(jnp.int32), peptide_emb)
    mhc_out = _mhc_matmul(mhc_x.astype(jnp.int32), mhc_emb)
    masks = msk.reshape(B, CORE_LEN).astype(jnp.bool_)
    return (pep_out, masks, mhc_out)


# speedup vs baseline: 1.8267x; 1.0834x over previous
import functools
import jax
import jax.numpy as jnp
from jax import lax
from jax.experimental import pallas as pl
from jax.experimental.pallas import tpu as pltpu
from jax.experimental.pallas import tpu_sc as plsc

B = 4096
PEP_LEN = 21
MHC_LEN = 34
CORE_LEN = 15
EMB = 128
PAD = 3
VOCAB = 30

NC = 2
NS = 16
NW = NC * NS

ROWS_W = B // NW
MSK_W = ROWS_W * CORE_LEN

GP = 4
NBUF = 4
PEP_ITERS = ROWS_W // (GP * NBUF)

_mesh = plsc.VectorSubcoreMesh(core_axis_name="c", subcore_axis_name="s")


@functools.partial(
    pl.kernel,
    mesh=_mesh,
    out_type=[
        jax.ShapeDtypeStruct((B, PEP_LEN, EMB), jnp.float32),
        jax.ShapeDtypeStruct((B * CORE_LEN,), jnp.int32),
    ],
    scratch_types=[
        pltpu.VMEM((ROWS_W, PEP_LEN), jnp.int32),
        pltpu.VMEM((MSK_W + 16,), jnp.int32),
        pltpu.VMEM_SHARED((VOCAB, EMB), jnp.float32),
    ]
    + [pltpu.VMEM((GP, PEP_LEN, EMB), jnp.float32) for _ in range(NBUF)]
    + [pltpu.SemaphoreType.DMA for _ in range(2 * NBUF + 1)],
)
def _pep_lookup(pep_x2, pep_tab, pep_out, msk_out,
                pep_idx2_v, msk_v, pep_tab_v, *bufs_and_sems):
    bufs = list(bufs_and_sems[:NBUF])
    gsems = list(bufs_and_sems[NBUF:2 * NBUF])
    osems = list(bufs_and_sems[2 * NBUF:3 * NBUF])
    msem = bufs_and_sems[3 * NBUF]

    wid = lax.axis_index("s") * NC + lax.axis_index("c")
    row0 = wid * ROWS_W

    pltpu.sync_copy(pep_x2.at[pl.ds(row0, ROWS_W)], pep_idx2_v)

    @pl.when(lax.axis_index("s") == 0)
    def _stage_table():
        pltpu.sync_copy(pep_tab, pep_tab_v)

    plsc.subcore_barrier()

    def mask_body(b, carry):
        mv = pep_idx2_v[b, pl.ds(PAD, 16)]
        m = jnp.where(mv != jnp.zeros((16,), jnp.int32),
                      jnp.ones((16,), jnp.int32),
                      jnp.zeros((16,), jnp.int32))
        msk_v[pl.ds(b * CORE_LEN, 16)] = m
        return carry

    lax.fori_loop(0, ROWS_W, mask_body, 0)
    mcopy = pltpu.async_copy(msk_v.at[pl.ds(0, MSK_W)],
                             msk_out.at[pl.ds(wid * MSK_W, MSK_W)], msem)

    def body(gg, carry):
        g0 = gg * NBUF
        gcs = []
        for s in range(NBUF):
            b0 = (g0 + s) * GP
            cps = []
            for r in range(GP):
                cps.append(pltpu.async_copy(
                    pep_tab_v.at[pep_idx2_v.at[b0 + r]], bufs[s].at[r],
                    gsems[s]))
            gcs.append(cps)
        ocs = []
        for s in range(NBUF):
            for cp in gcs[s]:
                cp.wait()
            b0 = (g0 + s) * GP
            ocs.append(pltpu.async_copy(
                bufs[s], pep_out.at[pl.ds(row0 + b0, GP)], osems[s]))
        for oc in ocs:
            oc.wait()
        return carry

    lax.fori_loop(0, PEP_ITERS, body, 0)
    mcopy.wait()


GM = 2
MHC_ITERS = ROWS_W // (GM * NBUF)


@functools.partial(
    pl.kernel,
    mesh=_mesh,
    out_type=jax.ShapeDtypeStruct((B, MHC_LEN, EMB), jnp.float32),
    scratch_types=[
        pltpu.VMEM((ROWS_W, MHC_LEN), jnp.int32),
        pltpu.VMEM_SHARED((VOCAB, EMB), jnp.float32),
    ]
    + [pltpu.VMEM((GM, MHC_LEN, EMB), jnp.float32) for _ in range(NBUF)]
    + [pltpu.SemaphoreType.DMA for _ in range(2 * NBUF)],
)
def _mhc_lookup(mhc_x2, mhc_tab, mhc_out,
                mhc_idx2_v, mhc_tab_v, *bufs_and_sems):
    bufs = list(bufs_and_sems[:NBUF])
    gsems = list(bufs_and_sems[NBUF:2 * NBUF])
    osems = list(bufs_and_sems[2 * NBUF:3 * NBUF])

    wid = lax.axis_index("s") * NC + lax.axis_index("c")
    row0 = wid * ROWS_W

    pltpu.sync_copy(mhc_x2.at[pl.ds(row0, ROWS_W)], mhc_idx2_v)

    @pl.when(lax.axis_index("s") == 0)
    def _stage_table():
        pltpu.sync_copy(mhc_tab, mhc_tab_v)

    plsc.subcore_barrier()

    def body(gg, carry):
        g0 = gg * NBUF
        gcs = []
        for s in range(NBUF):
            b0 = (g0 + s) * GM
            cps = []
            for r in range(GM):
                cps.append(pltpu.async_copy(
                    mhc_tab_v.at[mhc_idx2_v.at[b0 + r]], bufs[s].at[r],
                    gsems[s]))
            gcs.append(cps)
        ocs = []
        for s in range(NBUF):
            for cp in gcs[s]:
                cp.wait()
            b0 = (g0 + s) * GM
            ocs.append(pltpu.async_copy(
                bufs[s], mhc_out.at[pl.ds(row0 + b0, GM)], osems[s]))
        for oc in ocs:
            oc.wait()
        return carry

    lax.fori_loop(0, MHC_ITERS, body, 0)


def kernel(peptide_x, peptide_esm_x, mhc_x, peptide_emb, mhc_emb):
    del peptide_esm_x
    pep_out, msk = _pep_lookup(peptide_x.astype(jnp.int32), peptide_emb)
    mhc_out = _mhc_lookup(mhc_x.astype(jnp.int32), mhc_emb)
    masks = msk.reshape(B, CORE_LEN).astype(jnp.bool_)
    return (pep_out, masks, mhc_out)
